# baseline (speedup 1.0000x reference)
import jax
import jax.numpy as jnp
from jax import lax
from jax.experimental import pallas as pl
from jax.experimental.pallas import tpu as pltpu
from jax.experimental.pallas import tpu_sc as plsc

_L = 16
_NC, _NS = 2, 16
_NW = _NC * _NS
_SLAB = 128
_NBUF = 3


def _compute_slab(tp_v, xo, p):
    def per_pair(pr, _):
        la = pl.ds((pr * 2) * _L, _L)
        lb = pl.ds((pr * 2 + 1) * _L, _L)

        def amax(q, carry):
            ma, ia, mb, ib = carry
            va = xo[q, la]
            vb = xo[q, lb]
            ga = va > ma
            gb = vb > mb
            return (jnp.where(ga, va, ma), jnp.where(ga, q, ia),
                    jnp.where(gb, vb, mb), jnp.where(gb, q, ib))

        z = jnp.zeros((_L,), jnp.int32)
        _, ia, _, ib = lax.fori_loop(
            1, p, amax, (xo[0, la], z, xo[0, lb], z), unroll=8)

        aa0 = ia * p
        ab0 = ib * p
        ta0 = plsc.load_gather(tp_v, [aa0])
        tb0 = plsc.load_gather(tp_v, [ab0])
        xa0 = xo[0, la]
        xb0 = xo[0, lb]

        def gpass(q, carry):
            aa, ta, xa, ab, tb, xb = carry
            aa = aa + 1
            ab = ab + 1
            nta = plsc.load_gather(tp_v, [aa])
            ntb = plsc.load_gather(tp_v, [ab])
            nxa = xo[q, la]
            nxb = xo[q, lb]
            xo[q - 1, la] = jnp.maximum(xa * ta, 0.0)
            xo[q - 1, lb] = jnp.maximum(xb * tb, 0.0)
            return aa, nta, nxa, ab, ntb, nxb

        _, ta, xa, _, tb, xb = lax.fori_loop(
            1, p, gpass, (aa0, ta0, xa0, ab0, tb0, xb0), unroll=4)
        xo[p - 1, la] = jnp.maximum(xa * ta, 0.0)
        xo[p - 1, lb] = jnp.maximum(xb * tb, 0.0)
        return 0

    lax.fori_loop(0, _SLAB // (2 * _L), per_pair, 0)


def _sc_body(x_hbm, tp_hbm, o_hbm, tp_v, xo_v,
             in_s0, in_s1, in_s2, out_s0, out_s1, out_s2):
    b_total, p, c = x_hbm.shape
    bpw = b_total // _NW
    nslab = c // _SLAB
    nunits = bpw * nslab
    in_sems = (in_s0, in_s1, in_s2)
    out_sems = (out_s0, out_s1, out_s2)
    cid = lax.axis_index("c")
    sid = lax.axis_index("s")
    wid = sid * _NC + cid

    def unit_batch(u):
        return wid * bpw + u // nslab

    def start_in(u):
        g = u % nslab
        return pltpu.async_copy(
            x_hbm.at[unit_batch(u), :, pl.ds(g * _SLAB, _SLAB)],
            xo_v.at[u % _NBUF], in_sems[u % _NBUF])

    def start_out(u):
        g = u % nslab
        return pltpu.async_copy(
            xo_v.at[u % _NBUF],
            o_hbm.at[unit_batch(u), :, pl.ds(g * _SLAB, _SLAB)],
            out_sems[u % _NBUF])

    in_h = [None] * nunits
    out_h = [None] * nunits
    for u in range(min(2, nunits)):
        in_h[u] = start_in(u)
    for u in range(nunits):
        if u % nslab == 0:
            pltpu.sync_copy(tp_hbm.at[unit_batch(u)], tp_v)
        in_h[u].wait()
        _compute_slab(tp_v, xo_v.at[u % _NBUF], p)
        out_h[u] = start_out(u)
        if u + 2 < nunits:
            if u >= 1:
                out_h[u - 1].wait()
            in_h[u + 2] = start_in(u + 2)
    for u in range(max(0, nunits - 2), nunits):
        out_h[u].wait()


def kernel(input, t_p):
    b, h, w, c = input.shape
    p = h * w
    x = input.reshape(b, p, c)
    tp = t_p.reshape(b, p * p)
    mesh = plsc.VectorSubcoreMesh(core_axis_name="c", subcore_axis_name="s")
    run = pl.kernel(
        _sc_body,
        out_type=jax.ShapeDtypeStruct((b, p, c), jnp.float32),
        mesh=mesh,
        compiler_params=pltpu.CompilerParams(use_tc_tiling_on_sc=False,
                                             needs_layout_passes=False),
        scratch_types=[
            pltpu.VMEM((p * p,), jnp.float32),
            pltpu.VMEM((_NBUF, p, _SLAB), jnp.float32),
            pltpu.SemaphoreType.DMA,
            pltpu.SemaphoreType.DMA,
            pltpu.SemaphoreType.DMA,
            pltpu.SemaphoreType.DMA,
            pltpu.SemaphoreType.DMA,
            pltpu.SemaphoreType.DMA,
        ],
    )
    out = run(x, tp)
    return out.reshape(b, h, w, c)
